# baseline (speedup 1.0000x reference)
import jax
import jax.numpy as jnp
from jax import lax
from jax.experimental import pallas as pl
from jax.experimental.pallas import tpu as pltpu
from jax.experimental.pallas import tpu_sc as plsc

NUM_FEAT = 128
N_ROWS = 100000
NUM_CLASSES = 1000
C_PAD = 1024
BATCH = 1024
TEMP = 0.05
NC, NS = 2, 16
NW = NC * NS
CHUNK = 128
N_FULL = N_ROWS // CHUNK
N_TAIL = N_ROWS - N_FULL * CHUNK
N_BASE = N_FULL // NW
N_XTRA = N_FULL % NW
TGT_W = BATCH // NW
NB = 4


def _sc_body(feat_hbm, lab_hbm, lab2d_hbm, idx_hbm,
             sums_out, cnts_out, tgt_out,
             r0, r1, r2, r3, l0, l1, l2, l3, lblt_v, cnt_v, idxw_v, tgtw_v,
             rows16_v, zbuf_v,
             fs0, fs1, fs2, fs3, ls0, ls1, ls2, ls3, ss0, ss1, ss2, ss3,
             acc_sh):
    c = lax.axis_index("c")
    s = lax.axis_index("s")
    wid = s * NC + c

    rows = (r0, r1, r2, r3)
    lbl = (l0, l1, l2, l3)
    fsem = (fs0, fs1, fs2, fs3)
    lsem = (ls0, ls1, ls2, ls3)
    ssem = (ss0, ss1, ss2, ss3)

    zero16 = jnp.zeros((16,), jnp.float32)
    ones16 = jnp.ones((16,), jnp.float32)
    lane_iota = lax.iota(jnp.int32, 16)

    n_mine = N_BASE + jnp.where(wid < N_XTRA, 1, 0)

    def _fire(i, b):
        base = (wid + i * NW) * CHUNK
        pltpu.async_copy(feat_hbm.at[pl.ds(base, CHUNK), :], rows[b], fsem[b])
        pltpu.async_copy(lab_hbm.at[pl.ds(base, CHUNK)], lbl[b].at[0], lsem[b])

    def _wait_load(b):
        pltpu.make_async_copy(feat_hbm.at[pl.ds(0, CHUNK), :],
                              rows[b], fsem[b]).wait()
        pltpu.make_async_copy(lab_hbm.at[pl.ds(0, CHUNK)],
                              lbl[b].at[0], lsem[b]).wait()

    def _wait_scatter(b):
        pltpu.make_async_copy(rows[b], acc_sh.at[lbl[b].at[0]],
                              ssem[b]).wait()

    def _counts(b):
        for j in range(CHUNK // 16):
            lvec = lbl[b][0, pl.ds(j * 16, 16)]
            plsc.addupdate_scatter(cnt_v, [lane_iota * C_PAD + lvec], ones16)

    _fire(0, 0)
    _fire(1, 1)

    rz = C_PAD // NS

    def _zero_bufs(r, carry):
        for j in range(NUM_FEAT // 16):
            zbuf_v[r, pl.ds(j * 16, 16)] = zero16
        for j in range(16):
            cnt_v[pl.ds(r * 256 + j * 16, 16)] = zero16
        return carry

    lax.fori_loop(0, rz, _zero_bufs, 0)

    pltpu.sync_copy(zbuf_v, acc_sh.at[pl.ds(s * rz, rz), :])
    plsc.subcore_barrier()

    tb = wid * TGT_W
    pltpu.sync_copy(idx_hbm.at[pl.ds(tb, TGT_W)], idxw_v)
    for h in range(TGT_W // 16):
        iv = idxw_v[pl.ds(h * 16, 16)]
        rowv = lax.shift_right_logical(iv, 7)
        colv = lax.bitwise_and(iv, 127)
        pltpu.sync_copy(lab2d_hbm.at[rowv], rows16_v)
        tvec = plsc.load_gather(rows16_v, [lane_iota, colv])
        tgtw_v[pl.ds(h * 16, 16)] = tvec
    pltpu.sync_copy(tgtw_v, tgt_out.at[pl.ds(tb, TGT_W)])

    def _quad(g, carry):
        for sub in range(NB):
            i = NB * g + sub
            _wait_load(sub)
            pltpu.async_copy(rows[sub], acc_sh.at[lbl[sub].at[0]], ssem[sub],
                             add=True)
            _counts(sub)
            nxt = (sub + 2) % NB

            @pl.when(i >= 2)
            def _drain():
                _wait_scatter(nxt)

            @pl.when(i + 2 < n_mine)
            def _ahead():
                _fire(i + 2, nxt)
        return carry

    lax.fori_loop(0, N_BASE // NB, _quad, 0)

    @pl.when(wid < N_XTRA)
    def _extra():
        _wait_load(0)
        pltpu.async_copy(rows[0], acc_sh.at[lbl[0].at[0]], ssem[0], add=True)
        _counts(0)

    _wait_scatter(2)
    _wait_scatter(3)

    @pl.when(wid < N_XTRA)
    def _drain0():
        _wait_scatter(0)

    @pl.when(wid == NW - 1)
    def _tail():
        base = N_FULL * CHUNK
        pltpu.sync_copy(feat_hbm.at[pl.ds(base, N_TAIL), :],
                        r0.at[pl.ds(0, N_TAIL), :])
        pltpu.sync_copy(lab_hbm.at[pl.ds(base, N_TAIL)], lblt_v.at[0])
        pltpu.sync_copy(r0.at[pl.ds(0, N_TAIL), :],
                        acc_sh.at[lblt_v.at[0]], add=True)
        for j in range(N_TAIL // 16):
            lvec = lblt_v[0, pl.ds(j * 16, 16)]
            plsc.addupdate_scatter(cnt_v, [lane_iota * C_PAD + lvec], ones16)

    def _fold(k, carry):
        acc = cnt_v[pl.ds(k * 16, 16)]
        for j in range(1, 16):
            acc = acc + cnt_v[pl.ds(j * C_PAD + k * 16, 16)]
        cnt_v[pl.ds(k * 16, 16)] = acc
        return carry

    lax.fori_loop(0, C_PAD // 16, _fold, 0)
    pltpu.sync_copy(cnt_v.at[pl.ds(0, C_PAD)], cnts_out.at[wid])
    plsc.subcore_barrier()

    pltpu.sync_copy(acc_sh.at[pl.ds(s * rz, rz), :],
                    sums_out.at[c, pl.ds(s * rz, rz), :])


def _tc_body(sums_ref, cnts_ref, x_ref, y_ref, tgt_ref, back_ref, out_ref):
    f32 = jnp.float32
    cs = sums_ref[0] + sums_ref[1]
    c3 = cnts_ref[...]
    cntrow = jnp.sum(c3, axis=0, keepdims=True)
    cio = lax.broadcasted_iota(jnp.int32, (1, C_PAD), 1)
    mask = jnp.logical_and(cntrow > 0.0, cio < NUM_CLASSES).astype(f32)
    denom = mask * cntrow + (1.0 - mask)

    x = x_ref[...]
    sim = lax.dot_general(x, cs, (((1,), (1,)), ((), ())),
                          preferred_element_type=f32)
    sim = sim * (1.0 / TEMP) / denom
    e = jnp.exp(sim) * mask
    ssum = jnp.sum(e, axis=1, keepdims=True) + 1e-6
    tgt = tgt_ref[...]
    oh = (lax.broadcasted_iota(jnp.int32, (BATCH, C_PAD), 1)
          == tgt).astype(f32)
    p_t = jnp.sum(oh * e, axis=1, keepdims=True) / ssum
    focal = jnp.sum(-((1.0 - p_t) ** 4) * jnp.log(p_t + 1e-6)) / BATCH

    pickw = oh / denom
    picked = lax.dot_general(pickw, cs, (((1,), (0,)), ((), ())),
                             preferred_element_type=f32)
    y = y_ref[...]
    pn = picked / jnp.sqrt(jnp.sum(picked * picked, axis=1, keepdims=True))
    yn = y / jnp.sqrt(jnp.sum(y * y, axis=1, keepdims=True))
    memo = -jnp.sum(pn * yn) / BATCH
    xn = x / jnp.sqrt(jnp.sum(x * x, axis=1, keepdims=True))
    contra = -jnp.sum(xn * yn) / BATCH

    out_ref[0, 0] = focal + jnp.where(back_ref[0, 0] == 0, 0.0, memo + contra)


def kernel(inputs, another_inputs_full, indexes, back, features, labels):
    f32 = jnp.float32
    x = inputs.astype(f32)
    y = another_inputs_full.astype(f32)
    lab = labels.astype(jnp.int32)
    idx = indexes.astype(jnp.int32)
    feat = features.astype(f32)
    lab2d = jnp.pad(lab, (0, 128 * ((N_ROWS + 127) // 128) - N_ROWS)).reshape(-1, 128)

    mesh = plsc.VectorSubcoreMesh(core_axis_name="c", subcore_axis_name="s")
    sums, cnts, tgt = pl.kernel(
        _sc_body,
        out_type=[
            jax.ShapeDtypeStruct((NC, C_PAD, NUM_FEAT), f32),
            jax.ShapeDtypeStruct((NW, C_PAD), f32),
            jax.ShapeDtypeStruct((BATCH,), jnp.int32),
        ],
        mesh=mesh,
        compiler_params=pltpu.CompilerParams(needs_layout_passes=False),
        scratch_types=[
            pltpu.VMEM((CHUNK, NUM_FEAT), f32),
            pltpu.VMEM((CHUNK, NUM_FEAT), f32),
            pltpu.VMEM((CHUNK, NUM_FEAT), f32),
            pltpu.VMEM((CHUNK, NUM_FEAT), f32),
            pltpu.VMEM((1, CHUNK), jnp.int32),
            pltpu.VMEM((1, CHUNK), jnp.int32),
            pltpu.VMEM((1, CHUNK), jnp.int32),
            pltpu.VMEM((1, CHUNK), jnp.int32),
            pltpu.VMEM((1, N_TAIL), jnp.int32),
            pltpu.VMEM((16 * C_PAD,), f32),
            pltpu.VMEM((TGT_W,), jnp.int32),
            pltpu.VMEM((TGT_W,), jnp.int32),
            pltpu.VMEM((16, 128), jnp.int32),
            pltpu.VMEM((C_PAD // NS, NUM_FEAT), jnp.float32),
            pltpu.SemaphoreType.DMA,
            pltpu.SemaphoreType.DMA,
            pltpu.SemaphoreType.DMA,
            pltpu.SemaphoreType.DMA,
            pltpu.SemaphoreType.DMA,
            pltpu.SemaphoreType.DMA,
            pltpu.SemaphoreType.DMA,
            pltpu.SemaphoreType.DMA,
            pltpu.SemaphoreType.DMA,
            pltpu.SemaphoreType.DMA,
            pltpu.SemaphoreType.DMA,
            pltpu.SemaphoreType.DMA,
            pltpu.VMEM_SHARED((C_PAD, NUM_FEAT), f32),
        ],
    )(feat, lab, lab2d, idx)

    back_arr = jnp.asarray(back, jnp.int32).reshape(1, 1)
    out = pl.pallas_call(
        _tc_body,
        out_shape=jax.ShapeDtypeStruct((1, 1), f32),
        in_specs=[pl.BlockSpec(memory_space=pltpu.VMEM)] * 5
        + [pl.BlockSpec(memory_space=pltpu.SMEM)],
        out_specs=pl.BlockSpec(memory_space=pltpu.SMEM),
    )(sums, cnts, x, y, tgt.reshape(BATCH, 1), back_arr)
    return out[0, 0]


# speedup vs baseline: 1.0082x; 1.0082x over previous
import jax
import jax.numpy as jnp
from jax import lax
from jax.experimental import pallas as pl
from jax.experimental.pallas import tpu as pltpu
from jax.experimental.pallas import tpu_sc as plsc

NUM_FEAT = 128
N_ROWS = 100000
NUM_CLASSES = 1000
C_PAD = 1024
BATCH = 1024
TEMP = 0.05
NC, NS = 2, 16
NW = NC * NS
CHUNK = 128
N_FULL = N_ROWS // CHUNK
N_TAIL = N_ROWS - N_FULL * CHUNK
N_BASE = N_FULL // NW
N_XTRA = N_FULL % NW
TGT_W = BATCH // NW
NB = 4


def _sc_body(feat_hbm, lab_hbm, lab2d_hbm, idx_hbm,
             sums_out, cnts_out, tgt_out,
             r0, r1, r2, r3, l0, l1, l2, l3, lblt_v, cnt_v, idxw_v, tgtw_v,
             rows16_v, zbuf_v,
             fs0, fs1, fs2, fs3, ls0, ls1, ls2, ls3, ss0, ss1, ss2, ss3,
             acc_sh):
    c = lax.axis_index("c")
    s = lax.axis_index("s")
    wid = s * NC + c

    rows = (r0, r1, r2, r3)
    lbl = (l0, l1, l2, l3)
    fsem = (fs0, fs1, fs2, fs3)
    lsem = (ls0, ls1, ls2, ls3)
    ssem = (ss0, ss1, ss2, ss3)

    zero16 = jnp.zeros((16,), jnp.float32)
    ones16 = jnp.ones((16,), jnp.float32)
    lane_iota = lax.iota(jnp.int32, 16)

    n_mine = N_BASE + jnp.where(wid < N_XTRA, 1, 0)

    def _fire(i, b):
        base = (wid + i * NW) * CHUNK
        pltpu.async_copy(feat_hbm.at[pl.ds(base, CHUNK), :], rows[b], fsem[b])
        pltpu.async_copy(lab_hbm.at[pl.ds(base, CHUNK)], lbl[b].at[0], lsem[b])

    def _wait_load(b):
        pltpu.make_async_copy(feat_hbm.at[pl.ds(0, CHUNK), :],
                              rows[b], fsem[b]).wait()
        pltpu.make_async_copy(lab_hbm.at[pl.ds(0, CHUNK)],
                              lbl[b].at[0], lsem[b]).wait()

    def _wait_scatter(b):
        pltpu.make_async_copy(rows[b], acc_sh.at[lbl[b].at[0]],
                              ssem[b]).wait()

    def _counts(b):
        for j in range(CHUNK // 16):
            lvec = lbl[b][0, pl.ds(j * 16, 16)]
            plsc.addupdate_scatter(cnt_v, [lane_iota * C_PAD + lvec], ones16)

    _fire(0, 0)
    _fire(1, 1)

    rz = C_PAD // NS

    def _zero_bufs(r, carry):
        for j in range(NUM_FEAT // 16):
            zbuf_v[r, pl.ds(j * 16, 16)] = zero16
        for j in range(16):
            cnt_v[pl.ds(r * 256 + j * 16, 16)] = zero16
        return carry

    lax.fori_loop(0, rz, _zero_bufs, 0)

    pltpu.sync_copy(zbuf_v, acc_sh.at[pl.ds(s * rz, rz), :])
    plsc.subcore_barrier()

    tb = wid * TGT_W
    pltpu.sync_copy(idx_hbm.at[pl.ds(tb, TGT_W)], idxw_v)
    for h in range(TGT_W // 16):
        iv = idxw_v[pl.ds(h * 16, 16)]
        rowv = lax.shift_right_logical(iv, 7)
        colv = lax.bitwise_and(iv, 127)
        pltpu.sync_copy(lab2d_hbm.at[rowv], rows16_v)
        tvec = plsc.load_gather(rows16_v, [lane_iota, colv])
        tgtw_v[pl.ds(h * 16, 16)] = tvec
    pltpu.sync_copy(tgtw_v, tgt_out.at[pl.ds(tb, TGT_W)])

    def _quad(g, carry):
        for sub in range(NB):
            i = NB * g + sub
            _wait_load(sub)
            pltpu.async_copy(rows[sub], acc_sh.at[lbl[sub].at[0]], ssem[sub],
                             add=True)
            _counts(sub)
            nxt = (sub + 2) % NB

            @pl.when(i >= 2)
            def _drain():
                _wait_scatter(nxt)

            @pl.when(i + 2 < n_mine)
            def _ahead():
                _fire(i + 2, nxt)
        return carry

    lax.fori_loop(0, N_BASE // NB, _quad, 0)

    @pl.when(wid < N_XTRA)
    def _extra():
        _wait_load(0)
        pltpu.async_copy(rows[0], acc_sh.at[lbl[0].at[0]], ssem[0], add=True)
        _counts(0)

    @pl.when(wid == NW - 1)
    def _tail():
        base = N_FULL * CHUNK
        pltpu.sync_copy(feat_hbm.at[pl.ds(base, N_TAIL), :],
                        r0.at[pl.ds(0, N_TAIL), :])
        pltpu.sync_copy(lab_hbm.at[pl.ds(base, N_TAIL)], lblt_v.at[0])
        pltpu.sync_copy(r0.at[pl.ds(0, N_TAIL), :],
                        acc_sh.at[lblt_v.at[0]], add=True)
        for j in range(N_TAIL // 16):
            lvec = lblt_v[0, pl.ds(j * 16, 16)]
            plsc.addupdate_scatter(cnt_v, [lane_iota * C_PAD + lvec], ones16)

    def _fold(k, carry):
        acc = cnt_v[pl.ds(k * 16, 16)]
        for j in range(1, 16):
            acc = acc + cnt_v[pl.ds(j * C_PAD + k * 16, 16)]
        cnt_v[pl.ds(k * 16, 16)] = acc
        return carry

    lax.fori_loop(0, C_PAD // 16, _fold, 0)

    _wait_scatter(2)
    _wait_scatter(3)

    @pl.when(wid < N_XTRA)
    def _drain0():
        _wait_scatter(0)

    pltpu.sync_copy(cnt_v.at[pl.ds(0, C_PAD)], cnts_out.at[wid])
    plsc.subcore_barrier()

    pltpu.sync_copy(acc_sh.at[pl.ds(s * rz, rz), :],
                    sums_out.at[c, pl.ds(s * rz, rz), :])


def _tc_body(sums_ref, cnts_ref, x_ref, y_ref, tgt_ref, back_ref, out_ref):
    f32 = jnp.float32
    cs = sums_ref[0] + sums_ref[1]
    c3 = cnts_ref[...]
    cntrow = jnp.sum(c3, axis=0, keepdims=True)
    cio = lax.broadcasted_iota(jnp.int32, (1, C_PAD), 1)
    mask = jnp.logical_and(cntrow > 0.0, cio < NUM_CLASSES).astype(f32)
    denom = mask * cntrow + (1.0 - mask)

    x = x_ref[...]
    sim = lax.dot_general(x, cs, (((1,), (1,)), ((), ())),
                          preferred_element_type=f32)
    sim = sim * (1.0 / TEMP) / denom
    e = jnp.exp(sim) * mask
    ssum = jnp.sum(e, axis=1, keepdims=True) + 1e-6
    tgt = tgt_ref[...]
    oh = (lax.broadcasted_iota(jnp.int32, (BATCH, C_PAD), 1)
          == tgt).astype(f32)
    p_t = jnp.sum(oh * e, axis=1, keepdims=True) / ssum
    focal = jnp.sum(-((1.0 - p_t) ** 4) * jnp.log(p_t + 1e-6)) / BATCH

    pickw = oh / denom
    picked = lax.dot_general(pickw, cs, (((1,), (0,)), ((), ())),
                             preferred_element_type=f32)
    y = y_ref[...]
    pn = picked / jnp.sqrt(jnp.sum(picked * picked, axis=1, keepdims=True))
    yn = y / jnp.sqrt(jnp.sum(y * y, axis=1, keepdims=True))
    memo = -jnp.sum(pn * yn) / BATCH
    xn = x / jnp.sqrt(jnp.sum(x * x, axis=1, keepdims=True))
    contra = -jnp.sum(xn * yn) / BATCH

    out_ref[0, 0] = focal + jnp.where(back_ref[0, 0] == 0, 0.0, memo + contra)


def kernel(inputs, another_inputs_full, indexes, back, features, labels):
    f32 = jnp.float32
    x = inputs.astype(f32)
    y = another_inputs_full.astype(f32)
    lab = labels.astype(jnp.int32)
    idx = indexes.astype(jnp.int32)
    feat = features.astype(f32)
    lab2d = jnp.pad(lab, (0, 128 * ((N_ROWS + 127) // 128) - N_ROWS)).reshape(-1, 128)

    mesh = plsc.VectorSubcoreMesh(core_axis_name="c", subcore_axis_name="s")
    sums, cnts, tgt = pl.kernel(
        _sc_body,
        out_type=[
            jax.ShapeDtypeStruct((NC, C_PAD, NUM_FEAT), f32),
            jax.ShapeDtypeStruct((NW, C_PAD), f32),
            jax.ShapeDtypeStruct((BATCH,), jnp.int32),
        ],
        mesh=mesh,
        compiler_params=pltpu.CompilerParams(needs_layout_passes=False),
        scratch_types=[
            pltpu.VMEM((CHUNK, NUM_FEAT), f32),
            pltpu.VMEM((CHUNK, NUM_FEAT), f32),
            pltpu.VMEM((CHUNK, NUM_FEAT), f32),
            pltpu.VMEM((CHUNK, NUM_FEAT), f32),
            pltpu.VMEM((1, CHUNK), jnp.int32),
            pltpu.VMEM((1, CHUNK), jnp.int32),
            pltpu.VMEM((1, CHUNK), jnp.int32),
            pltpu.VMEM((1, CHUNK), jnp.int32),
            pltpu.VMEM((1, N_TAIL), jnp.int32),
            pltpu.VMEM((16 * C_PAD,), f32),
            pltpu.VMEM((TGT_W,), jnp.int32),
            pltpu.VMEM((TGT_W,), jnp.int32),
            pltpu.VMEM((16, 128), jnp.int32),
            pltpu.VMEM((C_PAD // NS, NUM_FEAT), jnp.float32),
            pltpu.SemaphoreType.DMA,
            pltpu.SemaphoreType.DMA,
            pltpu.SemaphoreType.DMA,
            pltpu.SemaphoreType.DMA,
            pltpu.SemaphoreType.DMA,
            pltpu.SemaphoreType.DMA,
            pltpu.SemaphoreType.DMA,
            pltpu.SemaphoreType.DMA,
            pltpu.SemaphoreType.DMA,
            pltpu.SemaphoreType.DMA,
            pltpu.SemaphoreType.DMA,
            pltpu.SemaphoreType.DMA,
            pltpu.VMEM_SHARED((C_PAD, NUM_FEAT), f32),
        ],
    )(feat, lab, lab2d, idx)

    back_arr = jnp.asarray(back, jnp.int32).reshape(1, 1)
    out = pl.pallas_call(
        _tc_body,
        out_shape=jax.ShapeDtypeStruct((1, 1), f32),
        in_specs=[pl.BlockSpec(memory_space=pltpu.VMEM)] * 5
        + [pl.BlockSpec(memory_space=pltpu.SMEM)],
        out_specs=pl.BlockSpec(memory_space=pltpu.SMEM),
    )(sums, cnts, x, y, tgt.reshape(BATCH, 1), back_arr)
    return out[0, 0]


# speedup vs baseline: 1.0092x; 1.0011x over previous
import jax
import jax.numpy as jnp
from jax import lax
from jax.experimental import pallas as pl
from jax.experimental.pallas import tpu as pltpu
from jax.experimental.pallas import tpu_sc as plsc

NUM_FEAT = 128
N_ROWS = 100000
NUM_CLASSES = 1000
C_PAD = 1024
BATCH = 1024
TEMP = 0.05
NC, NS = 2, 16
NW = NC * NS
CHUNK = 128
N_FULL = N_ROWS // CHUNK
N_TAIL = N_ROWS - N_FULL * CHUNK
N_BASE = N_FULL // NW
N_XTRA = N_FULL % NW
TGT_W = BATCH // NW
NB = 4


def _sc_body(feat_hbm, lab_hbm, lab2d_hbm, idx_hbm,
             sums_out, cnts_out, tgt_out,
             r0, r1, r2, r3, l0, l1, l2, l3, lblt_v, cnt_v, idxw_v, tgtw_v,
             rows16_v, zbuf_v,
             fs0, fs1, fs2, fs3, ls0, ls1, ls2, ls3, ss0, ss1, ss2, ss3,
             acc_sh):
    c = lax.axis_index("c")
    s = lax.axis_index("s")
    wid = s * NC + c

    rows = (r0, r1, r2, r3)
    lbl = (l0, l1, l2, l3)
    fsem = (fs0, fs1, fs2, fs3)
    lsem = (ls0, ls1, ls2, ls3)
    ssem = (ss0, ss1, ss2, ss3)

    zero16 = jnp.zeros((16,), jnp.float32)
    ones16 = jnp.ones((16,), jnp.float32)
    lane_iota = lax.iota(jnp.int32, 16)

    n_mine = N_BASE + jnp.where(wid < N_XTRA, 1, 0)

    def _fire(i, b):
        base = (wid + i * NW) * CHUNK
        pltpu.async_copy(feat_hbm.at[pl.ds(base, CHUNK), :], rows[b], fsem[b])
        pltpu.async_copy(lab_hbm.at[pl.ds(base, CHUNK)], lbl[b].at[0], lsem[b])

    def _wait_load(b):
        pltpu.make_async_copy(feat_hbm.at[pl.ds(0, CHUNK), :],
                              rows[b], fsem[b]).wait()
        pltpu.make_async_copy(lab_hbm.at[pl.ds(0, CHUNK)],
                              lbl[b].at[0], lsem[b]).wait()

    def _wait_scatter(b):
        pltpu.make_async_copy(rows[b], acc_sh.at[lbl[b].at[0]],
                              ssem[b]).wait()

    def _counts(b):
        for j in range(CHUNK // 16):
            lvec = lbl[b][0, pl.ds(j * 16, 16)]
            plsc.addupdate_scatter(cnt_v, [lane_iota * C_PAD + lvec], ones16)

    _fire(0, 0)
    _fire(1, 1)

    rz = C_PAD // NS

    def _zero_bufs(r, carry):
        for j in range(NUM_FEAT // 16):
            zbuf_v[r, pl.ds(j * 16, 16)] = zero16
        for j in range(16):
            cnt_v[pl.ds(r * 256 + j * 16, 16)] = zero16
        return carry

    lax.fori_loop(0, rz, _zero_bufs, 0)

    pltpu.sync_copy(zbuf_v, acc_sh.at[pl.ds(s * rz, rz), :])
    plsc.subcore_barrier()

    tb = wid * TGT_W
    pltpu.sync_copy(idx_hbm.at[pl.ds(tb, TGT_W)], idxw_v)
    for h in range(TGT_W // 16):
        iv = idxw_v[pl.ds(h * 16, 16)]
        rowv = lax.shift_right_logical(iv, 7)
        colv = lax.bitwise_and(iv, 127)
        pltpu.sync_copy(lab2d_hbm.at[rowv], rows16_v)
        tvec = plsc.load_gather(rows16_v, [lane_iota, colv])
        tgtw_v[pl.ds(h * 16, 16)] = tvec
    pltpu.sync_copy(tgtw_v, tgt_out.at[pl.ds(tb, TGT_W)])

    def _quad(g, carry):
        for sub in range(NB):
            i = NB * g + sub
            _wait_load(sub)
            pltpu.async_copy(rows[sub], acc_sh.at[lbl[sub].at[0]], ssem[sub],
                             add=True)
            _counts(sub)
            nxt = (sub + 2) % NB

            @pl.when(i >= 2)
            def _drain():
                _wait_scatter(nxt)

            @pl.when(i + 2 < n_mine)
            def _ahead():
                _fire(i + 2, nxt)
        return carry

    lax.fori_loop(0, N_BASE // NB, _quad, 0)

    @pl.when(wid < N_XTRA)
    def _extra():
        _wait_load(0)
        pltpu.async_copy(rows[0], acc_sh.at[lbl[0].at[0]], ssem[0], add=True)
        _counts(0)

    @pl.when(wid == NW - 1)
    def _tail():
        base = N_FULL * CHUNK
        pltpu.sync_copy(feat_hbm.at[pl.ds(base, N_TAIL), :],
                        r0.at[pl.ds(0, N_TAIL), :])
        pltpu.sync_copy(lab_hbm.at[pl.ds(base, N_TAIL)], lblt_v.at[0])
        pltpu.sync_copy(r0.at[pl.ds(0, N_TAIL), :],
                        acc_sh.at[lblt_v.at[0]], add=True)
        for j in range(N_TAIL // 16):
            lvec = lblt_v[0, pl.ds(j * 16, 16)]
            plsc.addupdate_scatter(cnt_v, [lane_iota * C_PAD + lvec], ones16)

    def _fold(k, carry):
        acc = cnt_v[pl.ds(k * 16, 16)]
        for j in range(1, 16):
            acc = acc + cnt_v[pl.ds(j * C_PAD + k * 16, 16)]
        cnt_v[pl.ds(k * 16, 16)] = acc
        return carry

    lax.fori_loop(0, C_PAD // 16, _fold, 0)

    _wait_scatter(2)
    _wait_scatter(3)

    @pl.when(wid < N_XTRA)
    def _drain0():
        _wait_scatter(0)

    pltpu.sync_copy(cnt_v.at[pl.ds(0, C_PAD)], cnts_out.at[wid])
    plsc.subcore_barrier()

    pltpu.sync_copy(acc_sh.at[pl.ds(s * rz, rz), :],
                    sums_out.at[pl.ds(c * C_PAD + s * rz, rz), :])


def _tc_body(sums_ref, cnts_ref, x_ref, y_ref, tgt_ref, back_ref, out_ref):
    f32 = jnp.float32
    s_all = sums_ref[...]
    cs = s_all[0:C_PAD] + s_all[C_PAD:2 * C_PAD]
    c3 = cnts_ref[...]
    cntrow = jnp.sum(c3, axis=0, keepdims=True)
    cio = lax.broadcasted_iota(jnp.int32, (1, C_PAD), 1)
    mask = jnp.logical_and(cntrow > 0.0, cio < NUM_CLASSES).astype(f32)
    denom = mask * cntrow + (1.0 - mask)

    x = x_ref[...]
    sim = lax.dot_general(x, cs, (((1,), (1,)), ((), ())),
                          preferred_element_type=f32)
    sim = sim * (1.0 / TEMP) / denom
    e = jnp.exp(sim) * mask
    ssum = jnp.sum(e, axis=1, keepdims=True) + 1e-6
    tgt = tgt_ref[...]
    oh = (lax.broadcasted_iota(jnp.int32, (BATCH, C_PAD), 1)
          == tgt).astype(f32)
    p_t = jnp.sum(oh * e, axis=1, keepdims=True) / ssum
    focal = jnp.sum(-((1.0 - p_t) ** 4) * jnp.log(p_t + 1e-6)) / BATCH

    pickw = oh / denom
    picked = lax.dot_general(pickw, cs, (((1,), (0,)), ((), ())),
                             preferred_element_type=f32)
    y = y_ref[...]
    pn = picked / jnp.sqrt(jnp.sum(picked * picked, axis=1, keepdims=True))
    yn = y / jnp.sqrt(jnp.sum(y * y, axis=1, keepdims=True))
    memo = -jnp.sum(pn * yn) / BATCH
    xn = x / jnp.sqrt(jnp.sum(x * x, axis=1, keepdims=True))
    contra = -jnp.sum(xn * yn) / BATCH

    out_ref[0, 0] = focal + jnp.where(back_ref[0, 0] == 0, 0.0, memo + contra)


def kernel(inputs, another_inputs_full, indexes, back, features, labels):
    f32 = jnp.float32
    x = inputs.astype(f32)
    y = another_inputs_full.astype(f32)
    lab = labels.astype(jnp.int32)
    idx = indexes.astype(jnp.int32)
    feat = features.astype(f32)
    lab2d = jnp.pad(lab, (0, 128 * ((N_ROWS + 127) // 128) - N_ROWS)).reshape(-1, 128)

    mesh = plsc.VectorSubcoreMesh(core_axis_name="c", subcore_axis_name="s")
    sums, cnts, tgt = pl.kernel(
        _sc_body,
        out_type=[
            jax.ShapeDtypeStruct((NC * C_PAD, NUM_FEAT), f32),
            jax.ShapeDtypeStruct((NW, C_PAD), f32),
            jax.ShapeDtypeStruct((BATCH,), jnp.int32),
        ],
        mesh=mesh,
        compiler_params=pltpu.CompilerParams(needs_layout_passes=False),
        scratch_types=[
            pltpu.VMEM((CHUNK, NUM_FEAT), f32),
            pltpu.VMEM((CHUNK, NUM_FEAT), f32),
            pltpu.VMEM((CHUNK, NUM_FEAT), f32),
            pltpu.VMEM((CHUNK, NUM_FEAT), f32),
            pltpu.VMEM((1, CHUNK), jnp.int32),
            pltpu.VMEM((1, CHUNK), jnp.int32),
            pltpu.VMEM((1, CHUNK), jnp.int32),
            pltpu.VMEM((1, CHUNK), jnp.int32),
            pltpu.VMEM((1, N_TAIL), jnp.int32),
            pltpu.VMEM((16 * C_PAD,), f32),
            pltpu.VMEM((TGT_W,), jnp.int32),
            pltpu.VMEM((TGT_W,), jnp.int32),
            pltpu.VMEM((16, 128), jnp.int32),
            pltpu.VMEM((C_PAD // NS, NUM_FEAT), jnp.float32),
            pltpu.SemaphoreType.DMA,
            pltpu.SemaphoreType.DMA,
            pltpu.SemaphoreType.DMA,
            pltpu.SemaphoreType.DMA,
            pltpu.SemaphoreType.DMA,
            pltpu.SemaphoreType.DMA,
            pltpu.SemaphoreType.DMA,
            pltpu.SemaphoreType.DMA,
            pltpu.SemaphoreType.DMA,
            pltpu.SemaphoreType.DMA,
            pltpu.SemaphoreType.DMA,
            pltpu.SemaphoreType.DMA,
            pltpu.VMEM_SHARED((C_PAD, NUM_FEAT), f32),
        ],
    )(feat, lab, lab2d, idx)

    back_arr = jnp.asarray(back, jnp.int32).reshape(1, 1)
    out = pl.pallas_call(
        _tc_body,
        out_shape=jax.ShapeDtypeStruct((1, 1), f32),
        in_specs=[pl.BlockSpec(memory_space=pltpu.VMEM)] * 5
        + [pl.BlockSpec(memory_space=pltpu.SMEM)],
        out_specs=pl.BlockSpec(memory_space=pltpu.SMEM),
    )(sums, cnts, x, y, tgt.reshape(BATCH, 1), back_arr)
    return out[0, 0]


# speedup vs baseline: 1.0409x; 1.0314x over previous
import jax
import jax.numpy as jnp
from jax import lax
from jax.experimental import pallas as pl
from jax.experimental.pallas import tpu as pltpu
from jax.experimental.pallas import tpu_sc as plsc

NUM_FEAT = 128
N_ROWS = 100000
NUM_CLASSES = 1000
C_PAD = 1024
BATCH = 1024
TEMP = 0.05
NC, NS = 2, 16
NW = NC * NS
CHUNK = 128
N_FULL = N_ROWS // CHUNK
N_TAIL = N_ROWS - N_FULL * CHUNK
N_BASE = N_FULL // NW
N_XTRA = N_FULL % NW
TGT_W = BATCH // NW
NB = 4


def _sc_body(feat_hbm, lab_hbm, lab2d_hbm, idx_hbm,
             sums_out, cnts_out, tgt_out,
             r0, r1, r2, r3, l0, l1, l2, l3, lblt_v, cnt_v, idxw_v, tgtw_v,
             rows16_v, zbuf_v,
             fs0, fs1, fs2, fs3, ls0, ls1, ls2, ls3, ss0, ss1, ss2, ss3,
             acc_sh):
    c = lax.axis_index("c")
    s = lax.axis_index("s")
    wid = s * NC + c

    rows = (r0, r1, r2, r3)
    lbl = (l0, l1, l2, l3)
    fsem = (fs0, fs1, fs2, fs3)
    lsem = (ls0, ls1, ls2, ls3)
    ssem = (ss0, ss1, ss2, ss3)

    zero16 = jnp.zeros((16,), jnp.float32)
    ones16 = jnp.ones((16,), jnp.float32)
    lane_iota = lax.iota(jnp.int32, 16)

    n_mine = N_BASE + jnp.where(wid < N_XTRA, 1, 0)

    def _fire(i, b):
        base = (wid + i * NW) * CHUNK
        pltpu.async_copy(feat_hbm.at[pl.ds(base, CHUNK), :], rows[b], fsem[b])
        pltpu.async_copy(lab_hbm.at[pl.ds(base, CHUNK)], lbl[b].at[0], lsem[b])

    def _wait_load(b):
        pltpu.make_async_copy(feat_hbm.at[pl.ds(0, CHUNK), :],
                              rows[b], fsem[b]).wait()
        pltpu.make_async_copy(lab_hbm.at[pl.ds(0, CHUNK)],
                              lbl[b].at[0], lsem[b]).wait()

    def _wait_scatter(b):
        pltpu.make_async_copy(rows[b], acc_sh.at[lbl[b].at[0]],
                              ssem[b]).wait()

    def _counts(b):
        for j in range(CHUNK // 16):
            lvec = lbl[b][0, pl.ds(j * 16, 16)]
            plsc.addupdate_scatter(cnt_v, [lane_iota * C_PAD + lvec], ones16)

    _fire(0, 0)
    _fire(1, 1)

    rz = C_PAD // NS

    def _zero_bufs(r, carry):
        for j in range(NUM_FEAT // 16):
            zbuf_v[r, pl.ds(j * 16, 16)] = zero16
        for j in range(16):
            cnt_v[pl.ds(r * 256 + j * 16, 16)] = zero16
        return carry

    lax.fori_loop(0, rz, _zero_bufs, 0)

    pltpu.sync_copy(zbuf_v, acc_sh.at[pl.ds(s * rz, rz), :])
    plsc.subcore_barrier()

    tb = wid * TGT_W
    pltpu.sync_copy(idx_hbm.at[pl.ds(tb, TGT_W)], idxw_v)
    for h in range(TGT_W // 16):
        iv = idxw_v[pl.ds(h * 16, 16)]
        rowv = lax.shift_right_logical(iv, 7)
        colv = lax.bitwise_and(iv, 127)
        pltpu.sync_copy(lab2d_hbm.at[rowv], rows16_v)
        tvec = plsc.load_gather(rows16_v, [lane_iota, colv])
        tgtw_v[pl.ds(h * 16, 16)] = tvec
    pltpu.sync_copy(tgtw_v, tgt_out.at[pl.ds(tb, TGT_W)])

    def _quad(g, carry):
        for sub in range(NB):
            i = NB * g + sub
            _wait_load(sub)
            pltpu.async_copy(rows[sub], acc_sh.at[lbl[sub].at[0]], ssem[sub],
                             add=True)
            _counts(sub)
            nxt = (sub + 2) % NB

            @pl.when(i >= 2)
            def _drain():
                _wait_scatter(nxt)

            @pl.when(i + 2 < n_mine)
            def _ahead():
                _fire(i + 2, nxt)
        return carry

    lax.fori_loop(0, N_BASE // NB, _quad, 0)

    @pl.when(wid < N_XTRA)
    def _extra():
        _wait_load(0)
        pltpu.async_copy(rows[0], acc_sh.at[lbl[0].at[0]], ssem[0], add=True)
        _counts(0)

    @pl.when(wid == NW - 1)
    def _tail():
        base = N_FULL * CHUNK
        pltpu.sync_copy(feat_hbm.at[pl.ds(base, N_TAIL), :],
                        r0.at[pl.ds(0, N_TAIL), :])
        pltpu.sync_copy(lab_hbm.at[pl.ds(base, N_TAIL)], lblt_v.at[0])
        pltpu.sync_copy(r0.at[pl.ds(0, N_TAIL), :],
                        acc_sh.at[lblt_v.at[0]], add=True)
        for j in range(N_TAIL // 16):
            lvec = lblt_v[0, pl.ds(j * 16, 16)]
            plsc.addupdate_scatter(cnt_v, [lane_iota * C_PAD + lvec], ones16)

    def _fold(k, carry):
        acc = cnt_v[pl.ds(k * 16, 16)]
        for j in range(1, 16):
            acc = acc + cnt_v[pl.ds(j * C_PAD + k * 16, 16)]
        cnt_v[pl.ds(k * 16, 16)] = acc
        return carry

    lax.fori_loop(0, C_PAD // 16, _fold, 0)

    _wait_scatter(2)
    _wait_scatter(3)

    @pl.when(wid < N_XTRA)
    def _drain0():
        _wait_scatter(0)

    pltpu.sync_copy(cnt_v.at[pl.ds(0, C_PAD)], cnts_out.at[wid])
    plsc.subcore_barrier()

    pltpu.sync_copy(acc_sh.at[pl.ds(s * rz, rz), :],
                    sums_out.at[pl.ds(c * C_PAD + s * rz, rz), :])


def _tc_body(sums_ref, cnts_ref, x_ref, y_ref, tgt_ref, back_ref, out_ref):
    f32 = jnp.float32
    s_all = sums_ref[...]
    cs = s_all[0:C_PAD] + s_all[C_PAD:2 * C_PAD]
    c3 = cnts_ref[...]
    cntrow = jnp.sum(c3, axis=0, keepdims=True)
    cio = lax.broadcasted_iota(jnp.int32, (1, C_PAD), 1)
    mask = jnp.logical_and(cntrow > 0.0, cio < NUM_CLASSES).astype(f32)
    denom = mask * cntrow + (1.0 - mask)

    x = x_ref[...]
    sim = lax.dot_general(x, cs, (((1,), (1,)), ((), ())),
                          preferred_element_type=f32)
    sim = sim * (1.0 / TEMP) / denom
    e = jnp.exp(sim) * mask
    ssum = jnp.sum(e, axis=1, keepdims=True) + 1e-6
    tgt = jnp.reshape(tgt_ref[...], (BATCH, 1))
    oh = (lax.broadcasted_iota(jnp.int32, (BATCH, C_PAD), 1)
          == tgt).astype(f32)
    p_t = jnp.sum(oh * e, axis=1, keepdims=True) / ssum
    focal = jnp.sum(-((1.0 - p_t) ** 4) * jnp.log(p_t + 1e-6)) / BATCH

    pickw = oh / denom
    picked = lax.dot_general(pickw, cs, (((1,), (0,)), ((), ())),
                             preferred_element_type=f32)
    y = y_ref[...]
    pn = picked / jnp.sqrt(jnp.sum(picked * picked, axis=1, keepdims=True))
    yn = y / jnp.sqrt(jnp.sum(y * y, axis=1, keepdims=True))
    memo = -jnp.sum(pn * yn) / BATCH
    xn = x / jnp.sqrt(jnp.sum(x * x, axis=1, keepdims=True))
    contra = -jnp.sum(xn * yn) / BATCH

    out_ref[0, 0] = focal + jnp.where(back_ref[0, 0] == 0, 0.0, memo + contra)


def kernel(inputs, another_inputs_full, indexes, back, features, labels):
    f32 = jnp.float32
    x = inputs.astype(f32)
    y = another_inputs_full.astype(f32)
    lab = labels.astype(jnp.int32)
    idx = indexes.astype(jnp.int32)
    feat = features.astype(f32)
    lab2d = jnp.pad(lab, (0, 128 * ((N_ROWS + 127) // 128) - N_ROWS)).reshape(-1, 128)

    mesh = plsc.VectorSubcoreMesh(core_axis_name="c", subcore_axis_name="s")
    sums, cnts, tgt = pl.kernel(
        _sc_body,
        out_type=[
            jax.ShapeDtypeStruct((NC * C_PAD, NUM_FEAT), f32),
            jax.ShapeDtypeStruct((NW, C_PAD), f32),
            jax.ShapeDtypeStruct((BATCH,), jnp.int32),
        ],
        mesh=mesh,
        compiler_params=pltpu.CompilerParams(needs_layout_passes=False),
        scratch_types=[
            pltpu.VMEM((CHUNK, NUM_FEAT), f32),
            pltpu.VMEM((CHUNK, NUM_FEAT), f32),
            pltpu.VMEM((CHUNK, NUM_FEAT), f32),
            pltpu.VMEM((CHUNK, NUM_FEAT), f32),
            pltpu.VMEM((1, CHUNK), jnp.int32),
            pltpu.VMEM((1, CHUNK), jnp.int32),
            pltpu.VMEM((1, CHUNK), jnp.int32),
            pltpu.VMEM((1, CHUNK), jnp.int32),
            pltpu.VMEM((1, N_TAIL), jnp.int32),
            pltpu.VMEM((16 * C_PAD,), f32),
            pltpu.VMEM((TGT_W,), jnp.int32),
            pltpu.VMEM((TGT_W,), jnp.int32),
            pltpu.VMEM((16, 128), jnp.int32),
            pltpu.VMEM((C_PAD // NS, NUM_FEAT), jnp.float32),
            pltpu.SemaphoreType.DMA,
            pltpu.SemaphoreType.DMA,
            pltpu.SemaphoreType.DMA,
            pltpu.SemaphoreType.DMA,
            pltpu.SemaphoreType.DMA,
            pltpu.SemaphoreType.DMA,
            pltpu.SemaphoreType.DMA,
            pltpu.SemaphoreType.DMA,
            pltpu.SemaphoreType.DMA,
            pltpu.SemaphoreType.DMA,
            pltpu.SemaphoreType.DMA,
            pltpu.SemaphoreType.DMA,
            pltpu.VMEM_SHARED((C_PAD, NUM_FEAT), f32),
        ],
    )(feat, lab, lab2d, idx)

    back_arr = jnp.asarray(back, jnp.int32).reshape(1, 1)
    out = pl.pallas_call(
        _tc_body,
        out_shape=jax.ShapeDtypeStruct((1, 1), f32),
        in_specs=[pl.BlockSpec(memory_space=pltpu.VMEM)] * 5
        + [pl.BlockSpec(memory_space=pltpu.SMEM)],
        out_specs=pl.BlockSpec(memory_space=pltpu.SMEM),
    )(sums, cnts, x, y, tgt.reshape(1, BATCH), back_arr)
    return out[0, 0]
